# baseline (speedup 1.0000x reference)
import functools

import jax
import jax.numpy as jnp
from jax import lax
from jax.experimental import pallas as pl
from jax.experimental.pallas import tpu as pltpu
from jax.experimental.pallas import tpu_sc as plsc

N = 100000
D = 128
G = 32
NEV = 512

NPAD = 100352
ROWS_PER_TILE = 3136
CHUNK = 112
NCHUNKS = ROWS_PER_TILE // CHUNK
TBL = 528
P1 = 1024
NB1 = NPAD // P1


def _mlp_body(x_ref, w1_ref, b1_ref, w2_ref, b2_ref, h_ref):
    x = x_ref[...].astype(jnp.bfloat16)
    t = jnp.maximum(
        jnp.dot(x, w1_ref[...], preferred_element_type=jnp.float32)
        + b1_ref[...], 0.0)
    h_ref[...] = (jnp.dot(t.astype(jnp.bfloat16), w2_ref[...],
                          preferred_element_type=jnp.float32)
                  + b2_ref[...])


def _mlp(x, W1, b1, W2, b2):
    const = lambda shape: pl.BlockSpec(shape, lambda i: (0,) * len(shape))
    return pl.pallas_call(
        _mlp_body,
        grid=(NB1,),
        in_specs=[
            pl.BlockSpec((P1, D), lambda i: (i, 0)),
            const((D, D)),
            const((1, D)),
            const((D, D)),
            const((1, D)),
        ],
        out_specs=pl.BlockSpec((P1, D), lambda i: (i, 0)),
        out_shape=jax.ShapeDtypeStruct((NPAD, D), jnp.float32),
        compiler_params=pltpu.CompilerParams(
            dimension_semantics=("arbitrary",),
        ),
    )(x, W1.astype(jnp.bfloat16), b1.reshape(1, D),
      W2.astype(jnp.bfloat16), b2.reshape(1, D))


def _sc_body(h_hbm, ev_hbm, ones_hbm, osum_hbm, ocnt_hbm,
             idx0, idx1, rows0, rows1, ones_v, zbuf,
             sem_h0, sem_h1, sem_e0, sem_e1, acc_sh, cnt_sh):
    cid = lax.axis_index("c")
    sid = lax.axis_index("s")
    wid = cid * 16 + sid

    pltpu.sync_copy(ones_hbm, ones_v)
    zv = jnp.zeros((16,), jnp.float32)
    for r in range(32):
        for c in range(D // 16):
            zbuf[r, pl.ds(c * 16, 16)] = zv

    zr = sid * 32
    pltpu.sync_copy(zbuf, acc_sh.at[pl.ds(zr, 32)])
    pltpu.sync_copy(zbuf, cnt_sh.at[pl.ds(zr, 32)])

    @pl.when(sid == 0)
    def _zero_dustbin():
        pltpu.sync_copy(zbuf.at[pl.ds(0, TBL - 512)],
                        acc_sh.at[pl.ds(512, TBL - 512)])
        pltpu.sync_copy(zbuf.at[pl.ds(0, TBL - 512)],
                        cnt_sh.at[pl.ds(512, TBL - 512)])

    plsc.subcore_barrier()

    base = wid * ROWS_PER_TILE
    idx = (idx0, idx1)
    rows = (rows0, rows1)
    sem_h = (sem_h0, sem_h1)
    sem_e = (sem_e0, sem_e1)

    def start(j):
        b = j % 2
        off = base + j * CHUNK
        ch = pltpu.async_copy(h_hbm.at[pl.ds(off, CHUNK), :], rows[b],
                              sem_h[b])
        ce = pltpu.async_copy(ev_hbm.at[pl.ds(off, CHUNK)], idx[b], sem_e[b])
        return ch, ce

    pend = start(0)
    for j in range(NCHUNKS):
        b = j % 2
        pend[0].wait()
        pend[1].wait()
        if j + 1 < NCHUNKS:
            pend = start(j + 1)
        pltpu.sync_copy(rows[b], acc_sh.at[idx[b]], add=True)
        pltpu.sync_copy(ones_v, cnt_sh.at[idx[b]], add=True)

    plsc.subcore_barrier()

    orow = sid * 32
    pltpu.sync_copy(acc_sh.at[pl.ds(orow, 32)],
                    osum_hbm.at[cid, pl.ds(orow, 32)])
    pltpu.sync_copy(cnt_sh.at[pl.ds(orow, 32)],
                    ocnt_hbm.at[cid, pl.ds(orow, 32)])


_sc_segment = functools.partial(
    pl.kernel,
    out_type=[
        jax.ShapeDtypeStruct((2, NEV, D), jnp.float32),
        jax.ShapeDtypeStruct((2, NEV, D), jnp.float32),
    ],
    mesh=plsc.VectorSubcoreMesh(core_axis_name="c", subcore_axis_name="s"),
    scratch_types=[
        pltpu.VMEM((CHUNK,), jnp.int32),
        pltpu.VMEM((CHUNK,), jnp.int32),
        pltpu.VMEM((CHUNK, D), jnp.float32),
        pltpu.VMEM((CHUNK, D), jnp.float32),
        pltpu.VMEM((CHUNK, D), jnp.float32),
        pltpu.VMEM((32, D), jnp.float32),
        pltpu.SemaphoreType.DMA,
        pltpu.SemaphoreType.DMA,
        pltpu.SemaphoreType.DMA,
        pltpu.SemaphoreType.DMA,
        pltpu.VMEM_SHARED((TBL, D), jnp.float32),
        pltpu.VMEM_SHARED((TBL, D), jnp.float32),
    ],
)(_sc_body)


def _combine_body(s_ref, c_ref, w3_ref, b3_ref, w4_ref, b4_ref, out_ref):
    gsum = s_ref[0] + s_ref[1]
    cnt = c_ref[0, :, 0:1] + c_ref[1, :, 0:1]
    gmean = gsum / jnp.maximum(cnt, 1.0)
    g = jnp.concatenate([gsum, gmean], axis=1)
    t = jnp.maximum(
        jnp.dot(g, w3_ref[...], preferred_element_type=jnp.float32)
        + b3_ref[...], 0.0)
    out_ref[...] = (jnp.dot(t, w4_ref[...],
                            preferred_element_type=jnp.float32)
                    + b4_ref[...])


def _combine(sums, cnts, W3, b3, W4, b4):
    return pl.pallas_call(
        _combine_body,
        out_shape=jax.ShapeDtypeStruct((NEV, G), jnp.float32),
    )(sums, cnts, W3, b3.reshape(1, 2 * D), W4, b4.reshape(1, G))


@jax.jit
def kernel(x, event, W1, b1, W2, b2, W3, b3, W4, b4):
    h = _mlp(x, W1, b1, W2, b2)
    ev_pad = jnp.concatenate(
        [event, jnp.full((NPAD - N,), NEV, jnp.int32)])
    ones = jnp.ones((CHUNK, D), jnp.float32)
    sums, cnts = _sc_segment(h, ev_pad, ones)
    return _combine(sums, cnts, W3, b3, W4, b4)


# speedup vs baseline: 1.1498x; 1.1498x over previous
import functools

import jax
import jax.numpy as jnp
from jax import lax
from jax.experimental import pallas as pl
from jax.experimental.pallas import tpu as pltpu
from jax.experimental.pallas import tpu_sc as plsc

N = 100000
D = 128
G = 32
NEV = 512

NPAD = 100352
NS = 2
HP = NPAD // NS
ROWS_PER_TILE = HP // 32
CHUNK = 112
NCHUNKS = ROWS_PER_TILE // CHUNK
TBL = 528
P1 = 1024
NB1 = NPAD // P1


def _mlp_body(x_ref, w1_ref, b1_ref, w2_ref, b2_ref, h_ref):
    x = x_ref[...].astype(jnp.bfloat16)
    t = jnp.maximum(
        jnp.dot(x, w1_ref[...], preferred_element_type=jnp.float32)
        + b1_ref[...], 0.0)
    h_ref[...] = (jnp.dot(t.astype(jnp.bfloat16), w2_ref[...],
                          preferred_element_type=jnp.float32)
                  + b2_ref[...])


def _mlp(x, W1, b1, W2, b2, start_blk):
    const = lambda shape: pl.BlockSpec(shape, lambda i: (0,) * len(shape))
    return pl.pallas_call(
        _mlp_body,
        grid=(NB1 // NS,),
        in_specs=[
            pl.BlockSpec((P1, D), lambda i: (start_blk + i, 0)),
            const((D, D)),
            const((1, D)),
            const((D, D)),
            const((1, D)),
        ],
        out_specs=pl.BlockSpec((P1, D), lambda i: (i, 0)),
        out_shape=jax.ShapeDtypeStruct((HP, D), jnp.float32),
        compiler_params=pltpu.CompilerParams(
            dimension_semantics=("arbitrary",),
        ),
    )(x, W1.astype(jnp.bfloat16), b1.reshape(1, D),
      W2.astype(jnp.bfloat16), b2.reshape(1, D))


def _sc_body(h_hbm, ev_hbm, ones_hbm, osum_hbm, ocnt_hbm,
             idx0, idx1, rows0, rows1, ones_v, zbuf,
             sem_h0, sem_h1, sem_e0, sem_e1, acc_sh, cnt_sh):
    cid = lax.axis_index("c")
    sid = lax.axis_index("s")
    wid = cid * 16 + sid

    pltpu.sync_copy(ones_hbm, ones_v)
    zv = jnp.zeros((16,), jnp.float32)
    for r in range(32):
        for c in range(D // 16):
            zbuf[r, pl.ds(c * 16, 16)] = zv

    zr = sid * 32
    pltpu.sync_copy(zbuf, acc_sh.at[pl.ds(zr, 32)])
    pltpu.sync_copy(zbuf, cnt_sh.at[pl.ds(zr, 32)])

    @pl.when(sid == 0)
    def _zero_dustbin():
        pltpu.sync_copy(zbuf.at[pl.ds(0, TBL - 512)],
                        acc_sh.at[pl.ds(512, TBL - 512)])
        pltpu.sync_copy(zbuf.at[pl.ds(0, TBL - 512)],
                        cnt_sh.at[pl.ds(512, TBL - 512)])

    plsc.subcore_barrier()

    base = wid * ROWS_PER_TILE
    idx = (idx0, idx1)
    rows = (rows0, rows1)
    sem_h = (sem_h0, sem_h1)
    sem_e = (sem_e0, sem_e1)

    def start(j):
        b = j % 2
        off = base + j * CHUNK
        ch = pltpu.async_copy(h_hbm.at[pl.ds(off, CHUNK), :], rows[b],
                              sem_h[b])
        ce = pltpu.async_copy(ev_hbm.at[pl.ds(off, CHUNK)], idx[b], sem_e[b])
        return ch, ce

    pend = start(0)
    for j in range(NCHUNKS):
        b = j % 2
        pend[0].wait()
        pend[1].wait()
        if j + 1 < NCHUNKS:
            pend = start(j + 1)
        pltpu.sync_copy(rows[b], acc_sh.at[idx[b]], add=True)
        pltpu.sync_copy(ones_v, cnt_sh.at[idx[b]], add=True)

    plsc.subcore_barrier()

    orow = sid * 32
    pltpu.sync_copy(acc_sh.at[pl.ds(orow, 32)],
                    osum_hbm.at[cid, pl.ds(orow, 32)])
    pltpu.sync_copy(cnt_sh.at[pl.ds(orow, 32)],
                    ocnt_hbm.at[cid, pl.ds(orow, 32)])


_sc_segment = functools.partial(
    pl.kernel,
    out_type=[
        jax.ShapeDtypeStruct((2, NEV, D), jnp.float32),
        jax.ShapeDtypeStruct((2, NEV, D), jnp.float32),
    ],
    mesh=plsc.VectorSubcoreMesh(core_axis_name="c", subcore_axis_name="s"),
    scratch_types=[
        pltpu.VMEM((CHUNK,), jnp.int32),
        pltpu.VMEM((CHUNK,), jnp.int32),
        pltpu.VMEM((CHUNK, D), jnp.float32),
        pltpu.VMEM((CHUNK, D), jnp.float32),
        pltpu.VMEM((CHUNK, D), jnp.float32),
        pltpu.VMEM((32, D), jnp.float32),
        pltpu.SemaphoreType.DMA,
        pltpu.SemaphoreType.DMA,
        pltpu.SemaphoreType.DMA,
        pltpu.SemaphoreType.DMA,
        pltpu.VMEM_SHARED((TBL, D), jnp.float32),
        pltpu.VMEM_SHARED((TBL, D), jnp.float32),
    ],
)(_sc_body)


def _combine_body(s0_ref, c0_ref, s1_ref, c1_ref,
                  w3_ref, b3_ref, w4_ref, b4_ref, out_ref):
    gsum = s0_ref[0] + s0_ref[1] + s1_ref[0] + s1_ref[1]
    cnt = (c0_ref[0, :, 0:1] + c0_ref[1, :, 0:1]
           + c1_ref[0, :, 0:1] + c1_ref[1, :, 0:1])
    gmean = gsum / jnp.maximum(cnt, 1.0)
    g = jnp.concatenate([gsum, gmean], axis=1)
    t = jnp.maximum(
        jnp.dot(g, w3_ref[...], preferred_element_type=jnp.float32)
        + b3_ref[...], 0.0)
    out_ref[...] = (jnp.dot(t, w4_ref[...],
                            preferred_element_type=jnp.float32)
                    + b4_ref[...])


def _combine(parts, W3, b3, W4, b4):
    return pl.pallas_call(
        _combine_body,
        out_shape=jax.ShapeDtypeStruct((NEV, G), jnp.float32),
    )(*parts, W3, b3.reshape(1, 2 * D), W4, b4.reshape(1, G))


@jax.jit
def kernel(x, event, W1, b1, W2, b2, W3, b3, W4, b4):
    ev_pad = jnp.concatenate(
        [event, jnp.full((NPAD - N,), NEV, jnp.int32)])
    ones = jnp.ones((CHUNK, D), jnp.float32)
    parts = []
    for s in range(NS):
        h = _mlp(x, W1, b1, W2, b2, s * (NB1 // NS))
        ev_s = lax.dynamic_slice(ev_pad, (s * HP,), (HP,))
        parts.extend(_sc_segment(h, ev_s, ones))
    return _combine(parts, W3, b3, W4, b4)
